# Initial kernel scaffold; baseline (speedup 1.0000x reference)
import jax
import jax.numpy as jnp
from jax.experimental import pallas as pl


def kernel(x, idx, out_size):
    raise NotImplementedError("write your pallas kernel here")


# speedup vs baseline: 4.9666x; 4.9666x over previous
import jax
import jax.numpy as jnp
from jax import lax
from jax.experimental import pallas as pl
from jax.experimental.pallas import tpu as pltpu
from jax.experimental.pallas import tpu_sc as plsc

N_IN = 500000
OUT = 1000000
C = 64
NW = 32
RPW = 31248
ZR = 496
NZ = RPW // ZR
TAIL = OUT - NW * RPW
IB = 100
GW = 8
WR = IB * GW
NG = N_IN // WR
SB = 48


def _sc_body(x_hbm, idx2_hbm, starts_hbm, out_hbm,
             zeros_v, idxw_v, xw_v, starts_s, sem_z):
    c = lax.axis_index("c")
    s = lax.axis_index("s")
    wid = s * 2 + c
    base = wid * RPW

    pltpu.sync_copy(starts_hbm, starts_s)

    zvec = jnp.zeros((16,), jnp.float32)

    @pl.loop(0, ZR)
    def _(r):
        for j in range(C // 16):
            zeros_v[r, pl.ds(j * 16, 16)] = zvec

    zcopies = [
        pltpu.async_copy(zeros_v, out_hbm.at[pl.ds(base + k * ZR, ZR)], sem_z)
        for k in range(NZ)
    ]
    for cp in zcopies:
        cp.wait()

    @pl.when(wid == NW - 1)
    def _():
        pltpu.async_copy(
            zeros_v.at[pl.ds(0, TAIL)],
            out_hbm.at[pl.ds(NW * RPW, TAIL)],
            sem_z,
        ).wait()

    sv = starts_s[pl.ds(wid, 16)]
    lo = sv[0]
    hi = sv[1]
    g0 = lo // WR
    g1 = (hi + WR - 1) // WR

    def win(g, carry):
        pltpu.sync_copy(idx2_hbm.at[pl.ds(g * GW, GW)], idxw_v)
        pltpu.sync_copy(x_hbm.at[pl.ds(g * WR, WR)], xw_v)
        for j in range(GW):
            pltpu.sync_copy(xw_v.at[pl.ds(j * IB, IB)],
                            out_hbm.at[idxw_v.at[j]])
        return carry

    lax.fori_loop(g0, g1, win, 0)


def kernel(x, idx, out_size):
    del out_size
    idx = idx.astype(jnp.int32)
    bounds = jnp.concatenate([
        jnp.arange(0, NW * RPW, RPW, dtype=jnp.int32),
        jnp.array([OUT], dtype=jnp.int32),
    ])
    starts = jnp.searchsorted(idx, bounds).astype(jnp.int32)
    starts = jnp.zeros((SB,), jnp.int32).at[: NW + 1].set(starts)
    idx2 = idx.reshape(NG * GW, IB)

    mesh = plsc.VectorSubcoreMesh(core_axis_name="c", subcore_axis_name="s")
    run = pl.kernel(
        _sc_body,
        out_type=jax.ShapeDtypeStruct((OUT, C), jnp.float32),
        mesh=mesh,
        compiler_params=pltpu.CompilerParams(use_tc_tiling_on_sc=False),
        scratch_types=[
            pltpu.VMEM((ZR, C), jnp.float32),
            pltpu.VMEM((GW, IB), jnp.int32),
            pltpu.VMEM((WR, C), jnp.float32),
            pltpu.VMEM((SB,), jnp.int32),
            pltpu.SemaphoreType.DMA,
        ],
    )
    return run(x, idx2, starts)
